# Initial kernel scaffold; baseline (speedup 1.0000x reference)
import jax
import jax.numpy as jnp
from jax.experimental import pallas as pl


def kernel(x, edge_index, edge_weight, W, b):
    raise NotImplementedError("write your pallas kernel here")


# speedup vs baseline: 1.2887x; 1.2887x over previous
import jax
import jax.numpy as jnp
from jax.experimental import pallas as pl
from jax.experimental.pallas import tpu as pltpu


def _mm_body(x_ref, w_ref, o_ref):
    o_ref[...] = jnp.dot(x_ref[...], w_ref[...],
                         preferred_element_type=jnp.float32)


def _matmul(x, W):
    n, cin = x.shape
    cout = W.shape[1]
    blk = 1000
    return pl.pallas_call(
        _mm_body,
        grid=(n // blk,),
        in_specs=[
            pl.BlockSpec((blk, cin), lambda i: (i, 0)),
            pl.BlockSpec((cin, cout), lambda i: (0, 0)),
        ],
        out_specs=pl.BlockSpec((blk, cout), lambda i: (i, 0)),
        out_shape=jax.ShapeDtypeStruct((n, cout), jnp.float32),
    )(x, W)


def kernel(x, edge_index, edge_weight, W, b):
    n = x.shape[0]
    src = edge_index[0]
    dst = edge_index[1]
    ew = edge_weight
    deg = jnp.ones((n,), jnp.float32).at[dst].add(ew)
    dinv = jax.lax.rsqrt(deg)
    h = _matmul(x, W)
    norm = dinv[src] * ew * dinv[dst]
    msgs = h[src] * norm[:, None]
    out = jnp.zeros((n, h.shape[1]), jnp.float32).at[dst].add(msgs)
    out = out + dinv[:, None] * dinv[:, None] * h
    return jax.nn.relu(out + b)


# speedup vs baseline: 17.8244x; 13.8313x over previous
import functools

import jax
import jax.numpy as jnp
from jax import lax
from jax.experimental import pallas as pl
from jax.experimental.pallas import tpu as pltpu
from jax.experimental.pallas import tpu_sc as plsc

NC = 2
NS = 16
NW = NC * NS
L = 16
K = 80


def _mm_body(x_ref, w_ref, o_ref):
    o_ref[...] = jnp.dot(x_ref[...], w_ref[...],
                         preferred_element_type=jnp.float32)


def _matmul(x, W):
    n, cin = x.shape
    cout = W.shape[1]
    blk = 1000
    return pl.pallas_call(
        _mm_body,
        grid=(n // blk,),
        in_specs=[
            pl.BlockSpec((blk, cin), lambda i: (i, 0)),
            pl.BlockSpec((cin, cout), lambda i: (0, 0)),
        ],
        out_specs=pl.BlockSpec((blk, cout), lambda i: (i, 0)),
        out_shape=jax.ShapeDtypeStruct((n, cout), jnp.float32),
    )(x, W)


def _deg_body(ept, npad, dst_hbm, ew_hbm, degp_hbm, dstv, ewv, dloc):
    c = lax.axis_index("c")
    s = lax.axis_index("s")
    wid = c * NS + s
    base = wid * ept
    pltpu.sync_copy(dst_hbm.at[pl.ds(base, ept)], dstv)
    pltpu.sync_copy(ew_hbm.at[pl.ds(base, ept)], ewv)

    zero = jnp.zeros((L,), jnp.float32)

    def zbody(i, _):
        dloc[pl.ds(i * L, L)] = zero
        return 0
    lax.fori_loop(0, npad // L, zbody, 0)

    def ebody(i, _):
        d = dstv[pl.ds(i * L, L)]
        w = ewv[pl.ds(i * L, L)]
        plsc.addupdate_scatter(dloc, [d], w)
        return 0
    lax.fori_loop(0, ept // L, ebody, 0)

    pltpu.sync_copy(dloc, degp_hbm.at[wid])


def _deg_partials(dst, ew, npad):
    e = dst.shape[0]
    ept = e // NW
    mesh = plsc.VectorSubcoreMesh(core_axis_name="c", subcore_axis_name="s", num_cores=NC, num_subcores=NS)
    body = functools.partial(_deg_body, ept, npad)
    f = pl.kernel(
        body,
        out_type=jax.ShapeDtypeStruct((NW, npad), jnp.float32),
        mesh=mesh,
        compiler_params=pltpu.CompilerParams(needs_layout_passes=False),
        scratch_types=[
            pltpu.VMEM((ept,), jnp.int32),
            pltpu.VMEM((ept,), jnp.float32),
            pltpu.VMEM((npad,), jnp.float32),
        ],
    )
    return f(dst, ew)


def _dinv_body(degp_ref, o_ref):
    deg = jnp.sum(degp_ref[...], axis=0) + 1.0
    o_ref[...] = lax.rsqrt(deg)


def _dinv(degp):
    npad = degp.shape[1]
    return pl.pallas_call(
        _dinv_body,
        out_shape=jax.ShapeDtypeStruct((npad,), jnp.float32),
    )(degp)


def _agg_body(ept, npad, ng, gsz, src_hbm, dst4_hbm, ew_hbm, h_hbm, dinv_hbm,
              outp_hbm, sg, dg, eg, dinvv, gbuf, scl, acc, sem):
    c = lax.axis_index("c")
    s = lax.axis_index("s")
    wid = c * NS + s
    ebase = wid * ept
    rpt = npad // NS
    nfeat = gbuf.shape[1]

    pltpu.sync_copy(dinv_hbm, dinvv)

    zero = jnp.zeros((L,), jnp.float32)
    for e in range(K):
        for j in range(nfeat // L):
            gbuf[e, pl.ds(j * L, L)] = zero
    for k in range(rpt // K):
        pltpu.sync_copy(gbuf, acc.at[pl.ds(s * rpt + k * K, K)])

    plsc.subcore_barrier()

    def group(g, _):
        gbase = ebase + g * gsz * K
        pltpu.sync_copy(src_hbm.at[pl.ds(gbase, gsz * K)], sg)
        pltpu.sync_copy(ew_hbm.at[pl.ds(gbase, gsz * K)], eg)
        pltpu.sync_copy(dst4_hbm.at[wid, g], dg)

        def chunk(i, _):
            pltpu.async_copy(h_hbm.at[sg.at[pl.ds(i * K, K)]], gbuf,
                             sem).wait()
            for e in range(K):
                idxv = jnp.full((L,), i * K + e, jnp.int32)
                sv = plsc.load_gather(sg, [idxv])
                ev = plsc.load_gather(eg, [idxv])
                dv = plsc.load_gather(dinvv, [sv])
                w = dv * ev
                for j in range(nfeat // L):
                    gbuf[e, pl.ds(j * L, L)] = gbuf[e, pl.ds(j * L, L)] * w
            pltpu.sync_copy(gbuf, acc.at[dg.at[i]], add=True)
            return 0

        lax.fori_loop(0, gsz, chunk, 0)
        return 0

    lax.fori_loop(0, ng, group, 0)

    plsc.subcore_barrier()

    pltpu.sync_copy(acc.at[pl.ds(s * rpt, rpt)],
                    outp_hbm.at[c, pl.ds(s * rpt, rpt)])


def _aggregate(src, dst, ew, h, dinv):
    e = src.shape[0]
    npad = dinv.shape[0]
    nfeat = h.shape[1]
    ept = e // NW
    nch = ept // K
    gsz = 25
    ng = nch // gsz
    dst4 = dst.reshape(NW, ng, gsz, K)
    mesh = plsc.VectorSubcoreMesh(core_axis_name="c", subcore_axis_name="s", num_cores=NC, num_subcores=NS)
    body = functools.partial(_agg_body, ept, npad, ng, gsz)
    f = pl.kernel(
        body,
        out_type=jax.ShapeDtypeStruct((NC, npad, nfeat), jnp.float32),
        mesh=mesh,
        compiler_params=pltpu.CompilerParams(needs_layout_passes=False),
        scratch_types=[
            pltpu.VMEM((gsz * K,), jnp.int32),
            pltpu.VMEM((gsz, K), jnp.int32),
            pltpu.VMEM((gsz * K,), jnp.float32),
            pltpu.VMEM((npad,), jnp.float32),
            pltpu.VMEM((K, nfeat), jnp.float32),
            pltpu.VMEM((K,), jnp.float32),
            pltpu.VMEM_SHARED((npad, nfeat), jnp.float32),
            pltpu.SemaphoreType.DMA,
        ],
    )
    return f(src, dst4, ew, h, dinv)


def _epi_body(p0_ref, p1_ref, h_ref, dinv_ref, b_ref, o_ref):
    dv = dinv_ref[...]
    agg = (p0_ref[...] + p1_ref[...]) * dv
    out = agg + h_ref[...] * (dv * dv) + b_ref[...]
    o_ref[...] = jnp.maximum(out, 0.0)


def _epilogue(p0, p1, h, dinv2d, b):
    n, nfeat = h.shape
    blk = 1000
    return pl.pallas_call(
        _epi_body,
        grid=(n // blk,),
        in_specs=[
            pl.BlockSpec((blk, nfeat), lambda i: (i, 0)),
            pl.BlockSpec((blk, nfeat), lambda i: (i, 0)),
            pl.BlockSpec((blk, nfeat), lambda i: (i, 0)),
            pl.BlockSpec((blk, 1), lambda i: (i, 0)),
            pl.BlockSpec((1, nfeat), lambda i: (0, 0)),
        ],
        out_specs=pl.BlockSpec((blk, nfeat), lambda i: (i, 0)),
        out_shape=jax.ShapeDtypeStruct((n, nfeat), jnp.float32),
    )(p0, p1, h, dinv2d, b)


def kernel(x, edge_index, edge_weight, W, b):
    n = x.shape[0]
    npad = ((n + NW * L - 1) // (NW * L)) * (NW * L)
    src = edge_index[0]
    dst = edge_index[1]
    ew = edge_weight

    h = _matmul(x, W)
    degp = _deg_partials(dst, ew, npad)
    dinv = _dinv(degp)
    outp = _aggregate(src, dst, ew, h, dinv)
    out = _epilogue(outp[0, :n], outp[1, :n], h,
                    dinv[:n, None], b[None, :])
    return out


# speedup vs baseline: 21.3601x; 1.1984x over previous
import functools

import jax
import jax.numpy as jnp
from jax import lax
from jax.experimental import pallas as pl
from jax.experimental.pallas import tpu as pltpu
from jax.experimental.pallas import tpu_sc as plsc

NC = 2
NS = 16
NW = NC * NS
L = 16
K = 40


def _mm_body(x_ref, w_ref, o_ref):
    o_ref[...] = jnp.dot(x_ref[...], w_ref[...],
                         preferred_element_type=jnp.float32)


def _matmul(x, W):
    n, cin = x.shape
    cout = W.shape[1]
    blk = 1000
    return pl.pallas_call(
        _mm_body,
        grid=(n // blk,),
        in_specs=[
            pl.BlockSpec((blk, cin), lambda i: (i, 0)),
            pl.BlockSpec((cin, cout), lambda i: (0, 0)),
        ],
        out_specs=pl.BlockSpec((blk, cout), lambda i: (i, 0)),
        out_shape=jax.ShapeDtypeStruct((n, cout), jnp.float32),
    )(x, W)


def _deg_body(ept, npad, dst_hbm, ew_hbm, degp_hbm, dstv, ewv, dloc):
    c = lax.axis_index("c")
    s = lax.axis_index("s")
    wid = c * NS + s
    base = wid * ept
    pltpu.sync_copy(dst_hbm.at[pl.ds(base, ept)], dstv)
    pltpu.sync_copy(ew_hbm.at[pl.ds(base, ept)], ewv)

    zero = jnp.zeros((L,), jnp.float32)

    def zbody(i, _):
        dloc[pl.ds(i * L, L)] = zero
        return 0
    lax.fori_loop(0, npad // L, zbody, 0)

    def ebody(i, _):
        d = dstv[pl.ds(i * L, L)]
        w = ewv[pl.ds(i * L, L)]
        plsc.addupdate_scatter(dloc, [d], w)
        return 0
    lax.fori_loop(0, ept // L, ebody, 0)

    pltpu.sync_copy(dloc, degp_hbm.at[wid])


def _deg_partials(dst, ew, npad):
    e = dst.shape[0]
    ept = e // NW
    mesh = plsc.VectorSubcoreMesh(core_axis_name="c", subcore_axis_name="s", num_cores=NC, num_subcores=NS)
    body = functools.partial(_deg_body, ept, npad)
    f = pl.kernel(
        body,
        out_type=jax.ShapeDtypeStruct((NW, npad), jnp.float32),
        mesh=mesh,
        compiler_params=pltpu.CompilerParams(needs_layout_passes=False),
        scratch_types=[
            pltpu.VMEM((ept,), jnp.int32),
            pltpu.VMEM((ept,), jnp.float32),
            pltpu.VMEM((npad,), jnp.float32),
        ],
    )
    return f(dst, ew)


def _dinv_body(degp_ref, o_ref):
    deg = jnp.sum(degp_ref[...], axis=0) + 1.0
    o_ref[...] = lax.rsqrt(deg)


def _dinv(degp):
    npad = degp.shape[1]
    return pl.pallas_call(
        _dinv_body,
        out_shape=jax.ShapeDtypeStruct((npad,), jnp.float32),
    )(degp)


def _agg_body(ept, npad, ng, gsz, src_hbm, dst4_hbm, ew_hbm, h_hbm, dinv_hbm,
              outp_hbm, sg, dg, eg, dinvv, gbufa, gbufb, acc,
              gsa, gsb, ssa, ssb):
    c = lax.axis_index("c")
    s = lax.axis_index("s")
    wid = c * NS + s
    ebase = wid * ept
    rpt = npad // NS
    nfeat = gbufa.shape[1]
    npairs = gsz // 2

    pltpu.sync_copy(dinv_hbm, dinvv)

    zero = jnp.zeros((L,), jnp.float32)
    for e in range(K):
        for j in range(nfeat // L):
            gbufa[e, pl.ds(j * L, L)] = zero
    for k in range(rpt // K):
        pltpu.sync_copy(gbufa, acc.at[pl.ds(s * rpt + k * K, K)])

    plsc.subcore_barrier()

    def scale_rows(buf, q):
        for e in range(K):
            idxv = jnp.full((L,), q * K + e, jnp.int32)
            sv = plsc.load_gather(sg, [idxv])
            ev = plsc.load_gather(eg, [idxv])
            dv = plsc.load_gather(dinvv, [sv])
            w = dv * ev
            for j in range(nfeat // L):
                buf[e, pl.ds(j * L, L)] = buf[e, pl.ds(j * L, L)] * w

    def group(g, _):
        gbase = ebase + g * gsz * K
        pltpu.sync_copy(src_hbm.at[pl.ds(gbase, gsz * K)], sg)
        pltpu.sync_copy(ew_hbm.at[pl.ds(gbase, gsz * K)], eg)
        pltpu.sync_copy(dst4_hbm.at[wid, g], dg)

        pltpu.async_copy(h_hbm.at[sg.at[pl.ds(0, K)]], gbufa, gsa)
        pltpu.async_copy(h_hbm.at[sg.at[pl.ds(K, K)]], gbufb, gsb)

        def pair(i, _):
            q0 = 2 * i
            q1 = 2 * i + 1
            pltpu.make_async_copy(h_hbm.at[sg.at[pl.ds(0, K)]], gbufa,
                                  gsa).wait()
            scale_rows(gbufa, q0)
            pltpu.async_copy(gbufa, acc.at[dg.at[q0]], ssa, add=True)
            pltpu.make_async_copy(h_hbm.at[sg.at[pl.ds(0, K)]], gbufb,
                                  gsb).wait()
            scale_rows(gbufb, q1)
            pltpu.async_copy(gbufb, acc.at[dg.at[q1]], ssb, add=True)
            pltpu.make_async_copy(gbufa, acc.at[dg.at[q0]], ssa).wait()

            @pl.when(i < npairs - 1)
            def _():
                pltpu.async_copy(h_hbm.at[sg.at[pl.ds((q0 + 2) * K, K)]],
                                 gbufa, gsa)

            pltpu.make_async_copy(gbufb, acc.at[dg.at[q1]], ssb).wait()

            @pl.when(i < npairs - 1)
            def _():
                pltpu.async_copy(h_hbm.at[sg.at[pl.ds((q1 + 2) * K, K)]],
                                 gbufb, gsb)
            return 0

        lax.fori_loop(0, npairs, pair, 0)
        return 0

    lax.fori_loop(0, ng, group, 0)

    plsc.subcore_barrier()

    pltpu.sync_copy(acc.at[pl.ds(s * rpt, rpt)],
                    outp_hbm.at[c, pl.ds(s * rpt, rpt)])


def _aggregate(src, dst, ew, h, dinv):
    e = src.shape[0]
    npad = dinv.shape[0]
    nfeat = h.shape[1]
    ept = e // NW
    nch = ept // K
    gsz = 50
    ng = nch // gsz
    dst4 = dst.reshape(NW, ng, gsz, K)
    mesh = plsc.VectorSubcoreMesh(core_axis_name="c", subcore_axis_name="s", num_cores=NC, num_subcores=NS)
    body = functools.partial(_agg_body, ept, npad, ng, gsz)
    f = pl.kernel(
        body,
        out_type=jax.ShapeDtypeStruct((NC, npad, nfeat), jnp.float32),
        mesh=mesh,
        compiler_params=pltpu.CompilerParams(needs_layout_passes=False),
        scratch_types=[
            pltpu.VMEM((gsz * K,), jnp.int32),
            pltpu.VMEM((gsz, K), jnp.int32),
            pltpu.VMEM((gsz * K,), jnp.float32),
            pltpu.VMEM((npad,), jnp.float32),
            pltpu.VMEM((K, nfeat), jnp.float32),
            pltpu.VMEM((K, nfeat), jnp.float32),
            pltpu.VMEM_SHARED((npad, nfeat), jnp.float32),
            pltpu.SemaphoreType.DMA,
            pltpu.SemaphoreType.DMA,
            pltpu.SemaphoreType.DMA,
            pltpu.SemaphoreType.DMA,
        ],
    )
    return f(src, dst4, ew, h, dinv)


def _epi_body(p0_ref, p1_ref, h_ref, dinv_ref, b_ref, o_ref):
    dv = dinv_ref[...]
    agg = (p0_ref[...] + p1_ref[...]) * dv
    out = agg + h_ref[...] * (dv * dv) + b_ref[...]
    o_ref[...] = jnp.maximum(out, 0.0)


def _epilogue(p0, p1, h, dinv2d, b):
    n, nfeat = h.shape
    blk = 1000
    return pl.pallas_call(
        _epi_body,
        grid=(n // blk,),
        in_specs=[
            pl.BlockSpec((blk, nfeat), lambda i: (i, 0)),
            pl.BlockSpec((blk, nfeat), lambda i: (i, 0)),
            pl.BlockSpec((blk, nfeat), lambda i: (i, 0)),
            pl.BlockSpec((blk, 1), lambda i: (i, 0)),
            pl.BlockSpec((1, nfeat), lambda i: (0, 0)),
        ],
        out_specs=pl.BlockSpec((blk, nfeat), lambda i: (i, 0)),
        out_shape=jax.ShapeDtypeStruct((n, nfeat), jnp.float32),
    )(p0, p1, h, dinv2d, b)


def kernel(x, edge_index, edge_weight, W, b):
    n = x.shape[0]
    npad = ((n + NW * L - 1) // (NW * L)) * (NW * L)
    src = edge_index[0]
    dst = edge_index[1]
    ew = edge_weight

    h = _matmul(x, W)
    degp = _deg_partials(dst, ew, npad)
    dinv = _dinv(degp)
    outp = _aggregate(src, dst, ew, h, dinv)
    out = _epilogue(outp[0, :n], outp[1, :n], h,
                    dinv[:n, None], b[None, :])
    return out


# speedup vs baseline: 25.2261x; 1.1810x over previous
import functools

import jax
import jax.numpy as jnp
from jax import lax
from jax.experimental import pallas as pl
from jax.experimental.pallas import tpu as pltpu
from jax.experimental.pallas import tpu_sc as plsc

NC = 2
NS = 16
NW = NC * NS
L = 16
K = 40


def _mm_body(x_ref, w_ref, o_ref):
    o_ref[...] = jnp.dot(x_ref[...], w_ref[...],
                         preferred_element_type=jnp.float32)


def _matmul(x, W):
    n, cin = x.shape
    cout = W.shape[1]
    blk = 1000
    return pl.pallas_call(
        _mm_body,
        grid=(n // blk,),
        in_specs=[
            pl.BlockSpec((blk, cin), lambda i: (i, 0)),
            pl.BlockSpec((cin, cout), lambda i: (0, 0)),
        ],
        out_specs=pl.BlockSpec((blk, cout), lambda i: (i, 0)),
        out_shape=jax.ShapeDtypeStruct((n, cout), jnp.float32),
    )(x, W)


def _deg_body(ept, npad, dst_hbm, ew_hbm, degp_hbm, dstv, ewv, dloc):
    c = lax.axis_index("c")
    s = lax.axis_index("s")
    wid = c * NS + s
    base = wid * ept
    pltpu.sync_copy(dst_hbm.at[pl.ds(base, ept)], dstv)
    pltpu.sync_copy(ew_hbm.at[pl.ds(base, ept)], ewv)

    zero = jnp.zeros((L,), jnp.float32)

    def zbody(i, _):
        dloc[pl.ds(i * L, L)] = zero
        return 0
    lax.fori_loop(0, npad // L, zbody, 0)

    def ebody(i, _):
        d = dstv[pl.ds(i * L, L)]
        w = ewv[pl.ds(i * L, L)]
        plsc.addupdate_scatter(dloc, [d], w)
        return 0
    lax.fori_loop(0, ept // L, ebody, 0)

    pltpu.sync_copy(dloc, degp_hbm.at[wid])


def _deg_partials(dst, ew, npad):
    e = dst.shape[0]
    ept = e // NW
    mesh = plsc.VectorSubcoreMesh(core_axis_name="c", subcore_axis_name="s", num_cores=NC, num_subcores=NS)
    body = functools.partial(_deg_body, ept, npad)
    f = pl.kernel(
        body,
        out_type=jax.ShapeDtypeStruct((NW, npad), jnp.float32),
        mesh=mesh,
        compiler_params=pltpu.CompilerParams(needs_layout_passes=False),
        scratch_types=[
            pltpu.VMEM((ept,), jnp.int32),
            pltpu.VMEM((ept,), jnp.float32),
            pltpu.VMEM((npad,), jnp.float32),
        ],
    )
    return f(dst, ew)


def _dinv_body(degp_ref, o_ref):
    deg = jnp.sum(degp_ref[...], axis=0) + 1.0
    o_ref[...] = lax.rsqrt(deg)


def _dinv(degp):
    npad = degp.shape[1]
    return pl.pallas_call(
        _dinv_body,
        out_shape=jax.ShapeDtypeStruct((npad,), jnp.float32),
    )(degp)


def _agg_body(ept, npad, ng, gsz, src_hbm, dst4_hbm, ew_hbm, h_hbm, dinv_hbm,
              outp_hbm, sg, dg, eg, dinvv, gbufa, gbufb, sclw, sclr, acc,
              sclsh, gsa, gsb, ssa, ssb):
    c = lax.axis_index("c")
    s = lax.axis_index("s")
    wid = c * NS + s
    ebase = wid * ept
    rpt = npad // NS
    nfeat = gbufa.shape[1]
    npairs = gsz // 2

    pltpu.sync_copy(dinv_hbm, dinvv)

    zero = jnp.zeros((L,), jnp.float32)
    for e in range(K):
        for j in range(nfeat // L):
            gbufa[e, pl.ds(j * L, L)] = zero
    for k in range(rpt // K):
        pltpu.sync_copy(gbufa, acc.at[pl.ds(s * rpt + k * K, K)])

    plsc.subcore_barrier()

    def scale_rows(buf, q):
        for e in range(K):
            w = plsc.load_gather(sclr, [jnp.full((L,), q * K + e,
                                                 jnp.int32)])
            for j in range(nfeat // L):
                buf[e, pl.ds(j * L, L)] = buf[e, pl.ds(j * L, L)] * w

    def group(g, _):
        gbase = ebase + g * gsz * K
        pltpu.sync_copy(src_hbm.at[pl.ds(gbase, gsz * K)], sg)
        pltpu.sync_copy(ew_hbm.at[pl.ds(gbase, gsz * K)], eg)
        pltpu.sync_copy(dst4_hbm.at[wid, g], dg)

        def sclk(k, _):
            iv = sg[pl.ds(k * L, L)]
            ev = eg[pl.ds(k * L, L)]
            sclw[pl.ds(k * L, L)] = plsc.load_gather(dinvv, [iv]) * ev
            return 0
        lax.fori_loop(0, gsz * K // L, sclk, 0)
        pltpu.sync_copy(sclw, sclsh.at[pl.ds(s * gsz * K, gsz * K)])
        pltpu.sync_copy(sclsh.at[pl.ds(s * gsz * K, gsz * K)], sclr)

        pltpu.async_copy(h_hbm.at[sg.at[pl.ds(0, K)]], gbufa, gsa)
        pltpu.async_copy(h_hbm.at[sg.at[pl.ds(K, K)]], gbufb, gsb)

        def pair(i, _):
            q0 = 2 * i
            q1 = 2 * i + 1
            pltpu.make_async_copy(h_hbm.at[sg.at[pl.ds(0, K)]], gbufa,
                                  gsa).wait()
            scale_rows(gbufa, q0)
            pltpu.async_copy(gbufa, acc.at[dg.at[q0]], ssa, add=True)
            pltpu.make_async_copy(h_hbm.at[sg.at[pl.ds(0, K)]], gbufb,
                                  gsb).wait()
            scale_rows(gbufb, q1)
            pltpu.async_copy(gbufb, acc.at[dg.at[q1]], ssb, add=True)
            pltpu.make_async_copy(gbufa, acc.at[dg.at[q0]], ssa).wait()

            @pl.when(i < npairs - 1)
            def _():
                pltpu.async_copy(h_hbm.at[sg.at[pl.ds((q0 + 2) * K, K)]],
                                 gbufa, gsa)

            pltpu.make_async_copy(gbufb, acc.at[dg.at[q1]], ssb).wait()

            @pl.when(i < npairs - 1)
            def _():
                pltpu.async_copy(h_hbm.at[sg.at[pl.ds((q1 + 2) * K, K)]],
                                 gbufb, gsb)
            return 0

        lax.fori_loop(0, npairs, pair, 0)
        return 0

    lax.fori_loop(0, ng, group, 0)

    plsc.subcore_barrier()

    pltpu.sync_copy(acc.at[pl.ds(s * rpt, rpt)],
                    outp_hbm.at[c, pl.ds(s * rpt, rpt)])


def _aggregate(src, dst, ew, h, dinv):
    e = src.shape[0]
    npad = dinv.shape[0]
    nfeat = h.shape[1]
    ept = e // NW
    nch = ept // K
    gsz = 50
    ng = nch // gsz
    dst4 = dst.reshape(NW, ng, gsz, K)
    mesh = plsc.VectorSubcoreMesh(core_axis_name="c", subcore_axis_name="s", num_cores=NC, num_subcores=NS)
    body = functools.partial(_agg_body, ept, npad, ng, gsz)
    f = pl.kernel(
        body,
        out_type=jax.ShapeDtypeStruct((NC, npad, nfeat), jnp.float32),
        mesh=mesh,
        compiler_params=pltpu.CompilerParams(needs_layout_passes=False),
        scratch_types=[
            pltpu.VMEM((gsz * K,), jnp.int32),
            pltpu.VMEM((gsz, K), jnp.int32),
            pltpu.VMEM((gsz * K,), jnp.float32),
            pltpu.VMEM((npad,), jnp.float32),
            pltpu.VMEM((K, nfeat), jnp.float32),
            pltpu.VMEM((K, nfeat), jnp.float32),
            pltpu.VMEM((gsz * K,), jnp.float32),
            pltpu.VMEM((gsz * K,), jnp.float32),
            pltpu.VMEM_SHARED((npad, nfeat), jnp.float32),
            pltpu.VMEM_SHARED((NS * gsz * K,), jnp.float32),
            pltpu.SemaphoreType.DMA,
            pltpu.SemaphoreType.DMA,
            pltpu.SemaphoreType.DMA,
            pltpu.SemaphoreType.DMA,
        ],
    )
    return f(src, dst4, ew, h, dinv)


def _epi_body(p0_ref, p1_ref, h_ref, dinv_ref, b_ref, o_ref):
    dv = dinv_ref[...]
    agg = (p0_ref[...] + p1_ref[...]) * dv
    out = agg + h_ref[...] * (dv * dv) + b_ref[...]
    o_ref[...] = jnp.maximum(out, 0.0)


def _epilogue(p0, p1, h, dinv2d, b):
    n, nfeat = h.shape
    blk = 1000
    return pl.pallas_call(
        _epi_body,
        grid=(n // blk,),
        in_specs=[
            pl.BlockSpec((blk, nfeat), lambda i: (i, 0)),
            pl.BlockSpec((blk, nfeat), lambda i: (i, 0)),
            pl.BlockSpec((blk, nfeat), lambda i: (i, 0)),
            pl.BlockSpec((blk, 1), lambda i: (i, 0)),
            pl.BlockSpec((1, nfeat), lambda i: (0, 0)),
        ],
        out_specs=pl.BlockSpec((blk, nfeat), lambda i: (i, 0)),
        out_shape=jax.ShapeDtypeStruct((n, nfeat), jnp.float32),
    )(p0, p1, h, dinv2d, b)


def kernel(x, edge_index, edge_weight, W, b):
    n = x.shape[0]
    npad = ((n + NW * L - 1) // (NW * L)) * (NW * L)
    src = edge_index[0]
    dst = edge_index[1]
    ew = edge_weight

    h = _matmul(x, W)
    degp = _deg_partials(dst, ew, npad)
    dinv = _dinv(degp)
    outp = _aggregate(src, dst, ew, h, dinv)
    out = _epilogue(outp[0, :n], outp[1, :n], h,
                    dinv[:n, None], b[None, :])
    return out
